# baseline (speedup 1.0000x reference)
import functools

import jax
import jax.numpy as jnp
from jax import lax
from jax.experimental import pallas as pl
from jax.experimental.pallas import tpu as pltpu
from jax.experimental.pallas import tpu_sc as plsc

MAX_POS = 1024
DIM = 64

try:
    _INFO = plsc.get_sparse_core_info()
    NC, NS, L = _INFO.num_cores, _INFO.num_subcores, _INFO.num_lanes
except Exception:
    NC, NS, L = 2, 16, 16
NW = NC * NS

DC = DIM // L
RB = NW // DC
CHUNK = 512


def _table_body(x_ref, y_ref, w_ref, b_ref, t_ref):
    wt = w_ref[...].T
    bias = b_ref[...] * 0.25
    xw = jnp.dot(x_ref[...], wt, preferred_element_type=jnp.float32) + bias
    yw = jnp.dot(y_ref[...], wt, preferred_element_type=jnp.float32) + bias
    t = jnp.concatenate([xw, yw], axis=0)
    t_ref[...] = t.reshape(2 * MAX_POS, DC, L).transpose(1, 0, 2).reshape(
        DC, 2 * MAX_POS * L
    )


def _build_table(x_table, y_table, W, b):
    return pl.pallas_call(
        _table_body,
        out_shape=jax.ShapeDtypeStruct((DC, 2 * MAX_POS * L), jnp.float32),
    )(x_table, y_table, W, b.reshape(1, DIM))


def _sc_body(rows_total, t_hbm, bb_hbm, out_hbm, tc_v, bb_v, out_v, sem):
    rw = rows_total // RB
    n_chunks = rw // CHUNK
    wid = lax.axis_index("s") * NC + lax.axis_index("c")
    rb = wid // DC
    dc = wid % DC
    base_row = rb * rw

    cp = pltpu.async_copy(t_hbm.at[dc], tc_v, sem)
    cp.wait()

    iota = lax.iota(jnp.int32, L)
    stride4 = iota * 4
    half = (jnp.zeros((L,), jnp.int32) + MAX_POS) * L

    def chunk_body(c, carry):
        row0 = base_row + c * CHUNK
        pltpu.sync_copy(bb_hbm.at[pl.ds(row0 * 4, 4 * CHUNK)], bb_v)

        def g_body(g, carry2):
            gvec = stride4 + g * (4 * L)
            rvec = iota + g * L
            ps = []
            for k in range(4):
                v = plsc.load_gather(bb_v, [gvec + k])
                f = jnp.clip(v * float(MAX_POS), 0.0, float(MAX_POS - 1))
                i = f.astype(jnp.int32) * L
                if k % 2:
                    i = i + half
                ps.append(i)
            for d in range(L):
                s = (
                    plsc.load_gather(tc_v, [ps[0] + d])
                    + plsc.load_gather(tc_v, [ps[1] + d])
                    + plsc.load_gather(tc_v, [ps[2] + d])
                    + plsc.load_gather(tc_v, [ps[3] + d])
                )
                plsc.store_scatter(
                    out_v, [rvec, jnp.full((L,), d, jnp.int32)], jnp.maximum(s, 0.0)
                )
            return carry2

        lax.fori_loop(0, CHUNK // L, g_body, 0)
        pltpu.sync_copy(out_v, out_hbm.at[pl.ds(row0, CHUNK), dc])
        return carry

    lax.fori_loop(0, n_chunks, chunk_body, 0)


def _lookup(t, bb_flat, rows_total):
    mesh = plsc.VectorSubcoreMesh(
        core_axis_name="c", subcore_axis_name="s", num_cores=NC, num_subcores=NS
    )
    f = pl.kernel(
        functools.partial(_sc_body, rows_total),
        out_type=jax.ShapeDtypeStruct((rows_total, DC, L), jnp.float32),
        mesh=mesh,
        scratch_types=[
            pltpu.VMEM((2 * MAX_POS * L,), jnp.float32),
            pltpu.VMEM((4 * CHUNK,), jnp.float32),
            pltpu.VMEM((CHUNK, L), jnp.float32),
            pltpu.SemaphoreType.DMA,
        ],
        compiler_params=pltpu.CompilerParams(
            use_tc_tiling_on_sc=False, needs_layout_passes=False
        ),
    )
    return f(t, bb_flat)


def kernel(gt_bboxes, x_table, y_table, W, b):
    B, N, _ = gt_bboxes.shape
    rows_total = B * N
    t = _build_table(x_table, y_table, W, b)
    bb_flat = gt_bboxes.reshape(rows_total * 4)
    out = _lookup(t, bb_flat, rows_total)
    return out.reshape(B, N, DIM)


# speedup vs baseline: 2.0560x; 2.0560x over previous
import functools

import jax
import jax.numpy as jnp
from jax import lax
from jax.experimental import pallas as pl
from jax.experimental.pallas import tpu as pltpu
from jax.experimental.pallas import tpu_sc as plsc

MAX_POS = 1024
DIM = 64

try:
    _INFO = plsc.get_sparse_core_info()
    NC, NS, L = _INFO.num_cores, _INFO.num_subcores, _INFO.num_lanes
except Exception:
    NC, NS, L = 2, 16, 16
NW = NC * NS

CHUNK = 128
GBLK = 128


def _table_body(x_ref, y_ref, w_ref, b_ref, t_ref):
    wt = w_ref[...].T
    bias = b_ref[...] * 0.25
    t_ref[0:MAX_POS, :] = (
        jnp.dot(x_ref[...], wt, preferred_element_type=jnp.float32) + bias
    )
    t_ref[MAX_POS : 2 * MAX_POS, :] = (
        jnp.dot(y_ref[...], wt, preferred_element_type=jnp.float32) + bias
    )


def _build_table(x_table, y_table, W, b):
    return pl.pallas_call(
        _table_body,
        out_shape=jax.ShapeDtypeStruct((2 * MAX_POS, DIM), jnp.float32),
    )(x_table, y_table, W, b.reshape(1, DIM))


def _sc_body(rows_total, t_hbm, bb_hbm, out_hbm, bb_v, idx_v, rows_v, out_v, sem):
    rw = rows_total // NW
    n_chunks = rw // CHUNK
    wid = lax.axis_index("s") * NC + lax.axis_index("c")
    base_row = wid * rw

    offs = (lax.iota(jnp.int32, L) % 2) * MAX_POS

    def chunk_body(c, carry):
        row0 = base_row + c * CHUNK
        pltpu.sync_copy(bb_hbm.at[pl.ds(row0 * 4, 4 * CHUNK)], bb_v)
        for j in range(4 * CHUNK // GBLK):
            for i in range(GBLK // L):
                v = bb_v[pl.ds(j * GBLK + i * L, L)]
                f = jnp.clip(v * float(MAX_POS), 0.0, float(MAX_POS - 1))
                idx_v[j, pl.ds(i * L, L)] = f.astype(jnp.int32) + offs
        copies = [
            pltpu.async_copy(
                t_hbm.at[idx_v.at[j]], rows_v.at[pl.ds(j * GBLK, GBLK)], sem
            )
            for j in range(4 * CHUNK // GBLK)
        ]
        for cp in copies:
            cp.wait()

        @plsc.parallel_loop(0, CHUNK, 1, unroll=8)
        def sum_body(r):
            for d in range(DIM // L):
                ds = pl.ds(d * L, L)
                s = (
                    rows_v[4 * r, ds]
                    + rows_v[4 * r + 1, ds]
                    + rows_v[4 * r + 2, ds]
                    + rows_v[4 * r + 3, ds]
                )
                out_v[r, ds] = jnp.maximum(s, 0.0)
        pltpu.sync_copy(out_v, out_hbm.at[pl.ds(row0, CHUNK)])
        return carry

    lax.fori_loop(0, n_chunks, chunk_body, 0)


def _lookup(t, bb_flat, rows_total):
    mesh = plsc.VectorSubcoreMesh(
        core_axis_name="c", subcore_axis_name="s", num_cores=NC, num_subcores=NS
    )
    f = pl.kernel(
        functools.partial(_sc_body, rows_total),
        out_type=jax.ShapeDtypeStruct((rows_total, DIM), jnp.float32),
        mesh=mesh,
        scratch_types=[
            pltpu.VMEM((4 * CHUNK,), jnp.float32),
            pltpu.VMEM((4 * CHUNK // GBLK, GBLK), jnp.int32),
            pltpu.VMEM((4 * CHUNK, DIM), jnp.float32),
            pltpu.VMEM((CHUNK, DIM), jnp.float32),
            pltpu.SemaphoreType.DMA,
        ],
        compiler_params=pltpu.CompilerParams(use_tc_tiling_on_sc=False),
    )
    return f(t, bb_flat)


def kernel(gt_bboxes, x_table, y_table, W, b):
    B, N, _ = gt_bboxes.shape
    rows_total = B * N
    t = _build_table(x_table, y_table, W, b)
    bb_flat = gt_bboxes.reshape(rows_total * 4)
    out = _lookup(t, bb_flat, rows_total)
    return out.reshape(B, N, DIM)


# speedup vs baseline: 2.2882x; 1.1129x over previous
import functools

import jax
import jax.numpy as jnp
from jax import lax
from jax.experimental import pallas as pl
from jax.experimental.pallas import tpu as pltpu
from jax.experimental.pallas import tpu_sc as plsc

MAX_POS = 1024
DIM = 64

try:
    _INFO = plsc.get_sparse_core_info()
    NC, NS, L = _INFO.num_cores, _INFO.num_subcores, _INFO.num_lanes
except Exception:
    NC, NS, L = 2, 16, 16
NW = NC * NS

CHUNK = 128
GBLK = 128


def _table_body(x_ref, y_ref, w_ref, b_ref, t_ref):
    wt = w_ref[...].T
    bias = b_ref[...] * 0.25
    t_ref[0:MAX_POS, :] = (
        jnp.dot(x_ref[...], wt, preferred_element_type=jnp.float32) + bias
    )
    t_ref[MAX_POS : 2 * MAX_POS, :] = (
        jnp.dot(y_ref[...], wt, preferred_element_type=jnp.float32) + bias
    )


def _build_table(x_table, y_table, W, b):
    return pl.pallas_call(
        _table_body,
        out_shape=jax.ShapeDtypeStruct((2 * MAX_POS, DIM), jnp.float32),
    )(x_table, y_table, W, b.reshape(1, DIM))


def _sc_body(
    rows_total,
    t_hbm,
    bb_hbm,
    out_hbm,
    bb_v,
    idx_v,
    rows_v,
    out_v,
    sem0,
    sem1,
    semw0,
    semw1,
):
    rw = rows_total // NW
    n_chunks = rw // CHUNK
    wid = lax.axis_index("s") * NC + lax.axis_index("c")
    base_row = wid * rw
    sems = (sem0, sem1)
    semws = (semw0, semw1)
    nj = 4 * CHUNK // GBLK

    offs = (lax.iota(jnp.int32, L) % 2) * MAX_POS

    def stage_and_fire(c, p):
        row0 = base_row + c * CHUNK
        pltpu.sync_copy(bb_hbm.at[pl.ds(row0 * 4, 4 * CHUNK)], bb_v)
        for j in range(nj):
            for i in range(GBLK // L):
                v = bb_v[pl.ds(j * GBLK + i * L, L)]
                f = jnp.clip(v * float(MAX_POS), 0.0, float(MAX_POS - 1))
                idx_v[p, j, pl.ds(i * L, L)] = f.astype(jnp.int32) + offs
        for j in range(nj):
            pltpu.async_copy(
                t_hbm.at[idx_v.at[p, j]],
                rows_v.at[p, pl.ds(j * GBLK, GBLK)],
                sems[p],
            )

    def consume(c, p, k):
        row0 = base_row + c * CHUNK
        for j in range(nj):
            pltpu.make_async_copy(
                t_hbm.at[idx_v.at[p, j]],
                rows_v.at[p, pl.ds(j * GBLK, GBLK)],
                sems[p],
            ).wait()

        @pl.when(k > 0)
        def _():
            prev0 = base_row + (c - 2) * CHUNK
            pltpu.make_async_copy(
                out_v.at[p], out_hbm.at[pl.ds(prev0, CHUNK)], semws[p]
            ).wait()

        @plsc.parallel_loop(0, CHUNK, 1, unroll=8)
        def sum_body(r):
            for d in range(DIM // L):
                ds = pl.ds(d * L, L)
                s = (
                    rows_v[p, 4 * r, ds]
                    + rows_v[p, 4 * r + 1, ds]
                    + rows_v[p, 4 * r + 2, ds]
                    + rows_v[p, 4 * r + 3, ds]
                )
                out_v[p, r, ds] = jnp.maximum(s, 0.0)

        pltpu.async_copy(out_v.at[p], out_hbm.at[pl.ds(row0, CHUNK)], semws[p])

    stage_and_fire(0, 0)

    def pair_body(k, carry):
        for p in range(2):
            c = 2 * k + p

            @pl.when(c + 1 < n_chunks)
            def _():
                stage_and_fire(c + 1, 1 - p)

            consume(c, p, k)
        return carry

    lax.fori_loop(0, n_chunks // 2, pair_body, 0)

    for p in range(2):
        last = base_row + (n_chunks - 2 + p) * CHUNK
        pltpu.make_async_copy(
            out_v.at[p], out_hbm.at[pl.ds(last, CHUNK)], semws[p]
        ).wait()


def _lookup(t, bb_flat, rows_total):
    mesh = plsc.VectorSubcoreMesh(
        core_axis_name="c", subcore_axis_name="s", num_cores=NC, num_subcores=NS
    )
    f = pl.kernel(
        functools.partial(_sc_body, rows_total),
        out_type=jax.ShapeDtypeStruct((rows_total, DIM), jnp.float32),
        mesh=mesh,
        scratch_types=[
            pltpu.VMEM((4 * CHUNK,), jnp.float32),
            pltpu.VMEM((2, 4 * CHUNK // GBLK, GBLK), jnp.int32),
            pltpu.VMEM((2, 4 * CHUNK, DIM), jnp.float32),
            pltpu.VMEM((2, CHUNK, DIM), jnp.float32),
            pltpu.SemaphoreType.DMA,
            pltpu.SemaphoreType.DMA,
            pltpu.SemaphoreType.DMA,
            pltpu.SemaphoreType.DMA,
        ],
        compiler_params=pltpu.CompilerParams(use_tc_tiling_on_sc=False),
    )
    return f(t, bb_flat)


def kernel(gt_bboxes, x_table, y_table, W, b):
    B, N, _ = gt_bboxes.shape
    rows_total = B * N
    t = _build_table(x_table, y_table, W, b)
    bb_flat = gt_bboxes.reshape(rows_total * 4)
    out = _lookup(t, bb_flat, rows_total)
    return out.reshape(B, N, DIM)


# speedup vs baseline: 2.5408x; 1.1104x over previous
import functools

import jax
import jax.numpy as jnp
from jax import lax
from jax.experimental import pallas as pl
from jax.experimental.pallas import tpu as pltpu
from jax.experimental.pallas import tpu_sc as plsc

MAX_POS = 1024
DIM = 64

try:
    _INFO = plsc.get_sparse_core_info()
    NC, NS, L = _INFO.num_cores, _INFO.num_subcores, _INFO.num_lanes
except Exception:
    NC, NS, L = 2, 16, 16
NW = NC * NS

CHUNK = 128
GBLK = 128


def _table_body(x_ref, y_ref, w_ref, b_ref, t_ref):
    wt = w_ref[...].T
    bias = b_ref[...] * 0.25
    t_ref[0:MAX_POS, :] = (
        jnp.dot(x_ref[...], wt, preferred_element_type=jnp.float32) + bias
    )
    t_ref[MAX_POS : 2 * MAX_POS, :] = (
        jnp.dot(y_ref[...], wt, preferred_element_type=jnp.float32) + bias
    )


def _build_table(x_table, y_table, W, b):
    return pl.pallas_call(
        _table_body,
        out_shape=jax.ShapeDtypeStruct((2 * MAX_POS, DIM), jnp.float32),
    )(x_table, y_table, W, b.reshape(1, DIM))


def _sc_body(
    rows_total,
    t_hbm,
    bb_hbm,
    out_hbm,
    bb_v,
    idx_v,
    rows_v,
    out_v,
    sem0,
    sem1,
    semw0,
    semw1,
):
    rw = rows_total // NW
    n_chunks = rw // CHUNK
    wid = lax.axis_index("s") * NC + lax.axis_index("c")
    base_row = wid * rw
    sems = (sem0, sem1)
    semws = (semw0, semw1)
    nj = 4 * CHUNK // GBLK

    offs = (lax.iota(jnp.int32, L) % 2) * MAX_POS

    def stage_and_fire(c, p):
        row0 = base_row + c * CHUNK
        pltpu.sync_copy(bb_hbm.at[pl.ds(row0 * 4, 4 * CHUNK)], bb_v)
        for j in range(nj):
            for i in range(GBLK // L):
                v = bb_v[pl.ds(j * GBLK + i * L, L)]
                f = jnp.clip(v * float(MAX_POS), 0.0, float(MAX_POS - 1))
                idx_v[p, j, pl.ds(i * L, L)] = f.astype(jnp.int32) + offs
        for j in range(nj):
            pltpu.async_copy(
                t_hbm.at[idx_v.at[p, j]],
                rows_v.at[p, pl.ds(j * GBLK, GBLK)],
                sems[p],
            )

    def consume(c, p, k):
        row0 = base_row + c * CHUNK
        for j in range(nj):
            pltpu.make_async_copy(
                t_hbm.at[idx_v.at[p, j]],
                rows_v.at[p, pl.ds(j * GBLK, GBLK)],
                sems[p],
            ).wait()

        @pl.when(k > 0)
        def _():
            prev0 = base_row + (c - 2) * CHUNK
            pltpu.make_async_copy(
                out_v.at[p], out_hbm.at[pl.ds(prev0, CHUNK)], semws[p]
            ).wait()

        @plsc.parallel_loop(0, CHUNK, 1, unroll=8)
        def sum_body(r):
            for h in range(DIM // (2 * L)):
                ds = pl.ds(h * 2 * L, 2 * L)
                a = [None] * 4
                bvs = [None] * 4
                for i in range(4):
                    a[i], bvs[i] = plsc.unpack(
                        rows_v[p, 4 * r + i, ds], format=plsc.PackFormat.INTERLEAVED
                    )
                sa = (a[0] + a[1]) + (a[2] + a[3])
                sb = (bvs[0] + bvs[1]) + (bvs[2] + bvs[3])
                out_v[p, r, pl.ds(h * 2 * L, L)] = jnp.maximum(sa, 0.0)
                out_v[p, r, pl.ds(h * 2 * L + L, L)] = jnp.maximum(sb, 0.0)

        pltpu.async_copy(out_v.at[p], out_hbm.at[pl.ds(row0, CHUNK)], semws[p])

    stage_and_fire(0, 0)

    def pair_body(k, carry):
        for p in range(2):
            c = 2 * k + p

            @pl.when(c + 1 < n_chunks)
            def _():
                stage_and_fire(c + 1, 1 - p)

            consume(c, p, k)
        return carry

    lax.fori_loop(0, n_chunks // 2, pair_body, 0)

    for p in range(2):
        last = base_row + (n_chunks - 2 + p) * CHUNK
        pltpu.make_async_copy(
            out_v.at[p], out_hbm.at[pl.ds(last, CHUNK)], semws[p]
        ).wait()


def _lookup(t, bb_flat, rows_total):
    mesh = plsc.VectorSubcoreMesh(
        core_axis_name="c", subcore_axis_name="s", num_cores=NC, num_subcores=NS
    )
    f = pl.kernel(
        functools.partial(_sc_body, rows_total),
        out_type=jax.ShapeDtypeStruct((rows_total, DIM), jnp.float32),
        mesh=mesh,
        scratch_types=[
            pltpu.VMEM((4 * CHUNK,), jnp.float32),
            pltpu.VMEM((2, 4 * CHUNK // GBLK, GBLK), jnp.int32),
            pltpu.VMEM((2, 4 * CHUNK, DIM), jnp.bfloat16),
            pltpu.VMEM((2, CHUNK, DIM), jnp.float32),
            pltpu.SemaphoreType.DMA,
            pltpu.SemaphoreType.DMA,
            pltpu.SemaphoreType.DMA,
            pltpu.SemaphoreType.DMA,
        ],
        compiler_params=pltpu.CompilerParams(
            use_tc_tiling_on_sc=False, needs_layout_passes=False
        ),
    )
    return f(t, bb_flat)


_PERM = []
for _h in range(DIM // 32):
    for _j in range(16):
        _PERM.extend((_h * 32 + _j, _h * 32 + 16 + _j))


def kernel(gt_bboxes, x_table, y_table, W, b):
    B, N, _ = gt_bboxes.shape
    rows_total = B * N
    t = _build_table(x_table, y_table, W, b)
    t = t[:, jnp.array(_PERM, jnp.int32)].astype(jnp.bfloat16)
    bb_flat = gt_bboxes.reshape(rows_total * 4)
    out = _lookup(t, bb_flat, rows_total)
    return out.reshape(B, N, DIM)
